# Initial kernel scaffold; baseline (speedup 1.0000x reference)
import jax
import jax.numpy as jnp
from jax.experimental import pallas as pl


def kernel(x, W):
    raise NotImplementedError("write your pallas kernel here")


# speedup vs baseline: 104.2676x; 104.2676x over previous
import math
import functools

import jax
import jax.numpy as jnp
from jax.experimental import pallas as pl
from jax.experimental.pallas import tpu as pltpu

_NUM_MODES = 7
_NUM_KEYS = 12
_KPAD = 16
_DECAY = math.exp(-0.0005)
_MAX_HEAT = 5.0
_T = 524288
_NC = 1024
_LC = _T // _NC
_CP = 128
_GA = _NC // _CP
_KS = 16
_JC = _LC // _KS
_BIG = 1e30
_A_CHUNK = math.exp(-0.0005 * _LC)


def _inc_from_notes(notes_row, dtype):
    keys = jnp.mod(notes_row, _NUM_KEYS)
    valid = notes_row > -10000
    rows = jax.lax.broadcasted_iota(jnp.int32, (_KPAD, notes_row.shape[1]), 0)
    return ((rows == keys) & valid).astype(dtype)


def _chunk_fold_kernel(x_ref, b_ref, c_ref):
    dt = b_ref.dtype

    def body(l, carry):
        b, c = carry
        inc = _inc_from_notes(x_ref[pl.ds(l, 1), :], dt)
        b = b * _DECAY + inc
        c = jnp.minimum(c * _DECAY + inc, _MAX_HEAT)
        return b, c

    b0 = jnp.zeros((_KPAD, _CP), dt)
    c0 = jnp.full((_KPAD, _CP), _BIG, dt)
    b, c = jax.lax.fori_loop(0, _LC, body, (b0, c0))
    b_ref[...] = b
    c_ref[...] = c


def _scan_kernel(b_ref, c_ref, h0_ref):
    dt = h0_ref.dtype
    av = jnp.full((_KPAD, _NC), _A_CHUNK, dt)
    bv = b_ref[...]
    cv = c_ref[...]
    col = jax.lax.broadcasted_iota(jnp.int32, (_KPAD, _NC), 1)

    def shift(x, o, fill):
        pad = jnp.full((_KPAD, o), fill, dt)
        return jnp.concatenate([pad, x[:, : _NC - o]], axis=1)

    o = 1
    while o < _NC:
        a1 = shift(av, o, 1.0)
        b1 = shift(bv, o, 0.0)
        c1 = shift(cv, o, _BIG)
        na = av * a1
        nb = av * b1 + bv
        nc = jnp.minimum(av * c1 + bv, cv)
        keep = col >= o
        av = jnp.where(keep, na, av)
        bv = jnp.where(keep, nb, bv)
        cv = jnp.where(keep, nc, cv)
        o *= 2

    h_end = jnp.minimum(bv, cv)
    h0_ref[...] = jnp.concatenate(
        [jnp.zeros((_KPAD, 1), dt), h_end[:, : _NC - 1]], axis=1)


def _replay_kernel(x_ref, h0_ref, w_ref, out_ref, h_ref):
    j = pl.program_id(1)

    @pl.when(j == 0)
    def _():
        h_ref[...] = h0_ref[...]

    notes = x_ref[pl.ds(j * _KS, _KS), :]
    h = h_ref[...]
    for k in range(_KS):
        inc = _inc_from_notes(notes[k : k + 1, :], h.dtype)
        h = jnp.minimum(h * _DECAY + inc, _MAX_HEAT)
        out_ref[:, k, :] = jax.lax.dot_general(
            h, w_ref[...], (((0,), (0,)), ((), ())),
            preferred_element_type=out_ref.dtype)
    h_ref[...] = h


@jax.jit
def kernel(x, W):
    dt = W.dtype
    x_t = x.reshape(_NC, _LC).T
    w_pad = jnp.zeros((_KPAD, _NUM_MODES * _NUM_KEYS), dt).at[:_NUM_KEYS].set(W)

    chunk_bc = pl.pallas_call(
        _chunk_fold_kernel,
        grid=(_GA,),
        in_specs=[pl.BlockSpec((_LC, _CP), lambda g: (0, g))],
        out_specs=[
            pl.BlockSpec((_KPAD, _CP), lambda g: (0, g)),
            pl.BlockSpec((_KPAD, _CP), lambda g: (0, g)),
        ],
        out_shape=[
            jax.ShapeDtypeStruct((_KPAD, _NC), dt),
            jax.ShapeDtypeStruct((_KPAD, _NC), dt),
        ],
        compiler_params=pltpu.CompilerParams(
            dimension_semantics=("parallel",)),
        name="chunk_fold",
    )
    b_arr, c_arr = chunk_bc(x_t)

    h_start = pl.pallas_call(
        _scan_kernel,
        out_shape=jax.ShapeDtypeStruct((_KPAD, _NC), dt),
        name="chunk_scan",
    )(b_arr, c_arr)

    out = pl.pallas_call(
        _replay_kernel,
        grid=(_GA, _JC),
        in_specs=[
            pl.BlockSpec((_LC, _CP), lambda g, j: (0, g)),
            pl.BlockSpec((_KPAD, _CP), lambda g, j: (0, g)),
            pl.BlockSpec((_KPAD, _NUM_MODES * _NUM_KEYS), lambda g, j: (0, 0)),
        ],
        out_specs=pl.BlockSpec(
            (_CP, _KS, _NUM_MODES * _NUM_KEYS), lambda g, j: (g, j, 0)),
        out_shape=jax.ShapeDtypeStruct((_NC, _LC, _NUM_MODES * _NUM_KEYS), dt),
        scratch_shapes=[pltpu.VMEM((_KPAD, _CP), dt)],
        compiler_params=pltpu.CompilerParams(
            dimension_semantics=("parallel", "arbitrary")),
        name="chunk_replay",
    )(x_t, h_start, w_pad)

    return out.reshape(1, _T, _NUM_MODES, _NUM_KEYS)
